# Initial kernel scaffold; baseline (speedup 1.0000x reference)
import jax
import jax.numpy as jnp
from jax.experimental import pallas as pl


def kernel(x, edge_index, edge_attr, W_dense, b_dense, W_mu, b_mu, W_logstd, b_logstd):
    raise NotImplementedError("write your pallas kernel here")


# speedup vs baseline: 23.3719x; 23.3719x over previous
import functools
import jax
import jax.numpy as jnp
from jax import lax
from jax.experimental import pallas as pl
from jax.experimental.pallas import tpu as pltpu
from jax.experimental.pallas import tpu_sc as plsc

CHUNK = 2048
SUB = 128
NSUB = CHUNK // SUB
CHUNK_B = 1024
NSUB_B = CHUNK_B // SUB
BN = 2000


def _deg_kernel(n, e_pad, dst2d, ewp, zn):
    mesh = plsc.VectorSubcoreMesh(core_axis_name="c", subcore_axis_name="s")
    per_w = e_pad // 32
    n_chunks = per_w // CHUNK

    @functools.partial(
        pl.kernel,
        out_type=jax.ShapeDtypeStruct((2, n), jnp.float32),
        mesh=mesh,
        scratch_types=[
            pltpu.VMEM((NSUB, SUB), jnp.int32),
            pltpu.VMEM((CHUNK,), jnp.float32),
            pltpu.VMEM_SHARED((n,), jnp.float32),
        ],
    )
    def body(dst_hbm, ew_hbm, z_hbm, out_hbm, dv, wv, deg_sh):
        c = lax.axis_index("c")
        s = lax.axis_index("s")
        wid = c * 16 + s

        @pl.when(s == 0)
        def _():
            pltpu.sync_copy(z_hbm, deg_sh)

        plsc.subcore_barrier()

        def chunk(t, carry):
            base = pl.multiple_of(wid * per_w + t * CHUNK, CHUNK)
            rbase = pl.multiple_of(base // SUB, NSUB)
            pltpu.sync_copy(dst_hbm.at[pl.ds(rbase, NSUB)], dv)
            pltpu.sync_copy(ew_hbm.at[pl.ds(base, CHUNK)], wv)
            for j in range(NSUB):
                pltpu.sync_copy(wv.at[pl.ds(j * SUB, SUB)],
                                deg_sh.at[dv.at[j]], add=True)
            return carry

        lax.fori_loop(0, n_chunks, chunk, 0)
        plsc.subcore_barrier()

        @pl.when(s == 0)
        def _():
            pltpu.sync_copy(deg_sh, out_hbm.at[c])

    return body(dst2d, ewp, zn)


def _agg_kernel(n, e_pad, g2r, src2d, dst2d, ewp, zn16):
    mesh = plsc.VectorSubcoreMesh(core_axis_name="c", subcore_axis_name="s")
    per_t = e_pad // 16
    n_chunks = per_t // CHUNK_B
    rows_a = ((n // 16 + 7) // 8) * 8

    @functools.partial(
        pl.kernel,
        out_type=jax.ShapeDtypeStruct((2, n, 16), jnp.float32),
        mesh=mesh,
        scratch_types=[
            pltpu.VMEM((NSUB_B, SUB), jnp.int32),
            pltpu.VMEM((NSUB_B, SUB), jnp.int32),
            pltpu.VMEM((NSUB_B, SUB), jnp.int32),
            pltpu.VMEM((CHUNK_B,), jnp.float32),
            pltpu.VMEM((CHUNK_B, 16), jnp.float32),
            pltpu.VMEM_SHARED((n, 16), jnp.float32),
        ],
        compiler_params=pltpu.CompilerParams(use_tc_tiling_on_sc=False),
    )
    def body(g_hbm, src_hbm, dst_hbm, ew_hbm, z_hbm, out_hbm,
             sv, iv, dv, wv, rows, acc_sh):
        c = lax.axis_index("c")
        s = lax.axis_index("s")
        coff = c * n

        @pl.when(s == 0)
        def _():
            pltpu.sync_copy(z_hbm, acc_sh)

        plsc.subcore_barrier()

        lane = lax.iota(jnp.int32, 16)

        def chunk(t, carry):
            base = pl.multiple_of(s * per_t + t * CHUNK_B, CHUNK_B)
            rbase = pl.multiple_of(base // SUB, NSUB_B)
            pltpu.sync_copy(src_hbm.at[pl.ds(rbase, NSUB_B)], sv)
            pltpu.sync_copy(dst_hbm.at[pl.ds(rbase, NSUB_B)], dv)
            pltpu.sync_copy(ew_hbm.at[pl.ds(base, CHUNK_B)], wv)

            for j in range(NSUB_B):
                def addoff(k, cc):
                    iv[j, pl.ds(k * 16, 16)] = sv[j, pl.ds(k * 16, 16)] + coff
                    return cc
                lax.fori_loop(0, SUB // 16, addoff, 0)

            for j in range(NSUB_B):
                pltpu.sync_copy(g_hbm.at[iv.at[j]],
                                rows.at[pl.ds(j * SUB, SUB)])

            def scale(g, cc):
                ewv = wv[pl.ds(g * 16, 16)]
                for e in range(16):
                    be = ewv.at[jnp.full((16,), e, jnp.int32)].get(
                        mode="promise_in_bounds")
                    r = g * 16 + e
                    rows[r, :] = rows[r, :] * be
                return cc
            lax.fori_loop(0, CHUNK_B // 16, scale, 0)

            for j in range(NSUB_B):
                pltpu.sync_copy(rows.at[pl.ds(j * SUB, SUB)],
                                acc_sh.at[dv.at[j]], add=True)
            return carry

        lax.fori_loop(0, n_chunks, chunk, 0)
        plsc.subcore_barrier()

        r0 = pl.multiple_of(s * rows_a, 8)
        rows_last = n - 15 * rows_a

        @pl.when(s < 15)
        def _():
            pltpu.sync_copy(acc_sh.at[pl.ds(r0, rows_a)],
                            out_hbm.at[c, pl.ds(r0, rows_a)])

        @pl.when(s == 15)
        def _():
            pltpu.sync_copy(acc_sh.at[pl.ds(r0, rows_last)],
                            out_hbm.at[c, pl.ds(r0, rows_last)])

    return body(g2r, src2d, dst2d, ewp, zn16)


def _dense_kernel(x, wdt, bd, wct, bcat, dpt):
    n = x.shape[0]
    grid = n // BN

    def body(x_ref, wdt_ref, bd_ref, wct_ref, bcat_ref, dpt_ref,
             g2_ref, s2_ref, dinv_ref):
        xb = x_ref[...]
        h = jnp.maximum(
            lax.dot_general(xb, wdt_ref[...], (((1,), (0,)), ((), ())),
                            precision=lax.Precision.HIGHEST,
                            preferred_element_type=jnp.float32)
            + bd_ref[...], 0.0)
        h2 = lax.dot_general(h, wct_ref[...], (((1,), (0,)), ((), ())),
                             precision=lax.Precision.HIGHEST,
                             preferred_element_type=jnp.float32)
        dp = dpt_ref[...]
        deg = dp[:, 0] + dp[:, 1] + 1.0
        dinv = lax.rsqrt(deg)
        dinv_ref[...] = dinv[:, None]
        g = h2 * dinv[:, None]
        sself = h2 * (dinv * dinv)[:, None]
        g2_ref[0] = g[:, :16]
        g2_ref[1] = g[:, 16:]
        s2_ref[0] = sself[:, :16] + bcat_ref[0][None, :]
        s2_ref[1] = sself[:, 16:] + bcat_ref[1][None, :]

    return pl.pallas_call(
        body,
        grid=(grid,),
        in_specs=[
            pl.BlockSpec((BN, 128), lambda i: (i, 0)),
            pl.BlockSpec((128, 32), lambda i: (0, 0)),
            pl.BlockSpec((1, 32), lambda i: (0, 0)),
            pl.BlockSpec((32, 32), lambda i: (0, 0)),
            pl.BlockSpec((2, 16), lambda i: (0, 0)),
            pl.BlockSpec((BN, 2), lambda i: (i, 0)),
        ],
        out_specs=[
            pl.BlockSpec((2, BN, 16), lambda i: (0, i, 0)),
            pl.BlockSpec((2, BN, 16), lambda i: (0, i, 0)),
            pl.BlockSpec((BN, 1), lambda i: (i, 0)),
        ],
        out_shape=[
            jax.ShapeDtypeStruct((2, n, 16), jnp.float32),
            jax.ShapeDtypeStruct((2, n, 16), jnp.float32),
            jax.ShapeDtypeStruct((n, 1), jnp.float32),
        ],
    )(x, wdt, bd, wct, bcat, dpt)


def _final_kernel(acc2, s2, dinv):
    n = acc2.shape[1]
    grid = n // BN

    def body(a_ref, s_ref, d_ref, mu_ref, ls_ref):
        a = a_ref[...]
        sv = s_ref[...]
        d = d_ref[...]
        mu_ref[...] = a[0] * d + sv[0]
        ls_ref[...] = a[1] * d + sv[1]

    return pl.pallas_call(
        body,
        grid=(grid,),
        in_specs=[
            pl.BlockSpec((2, BN, 16), lambda i: (0, i, 0)),
            pl.BlockSpec((2, BN, 16), lambda i: (0, i, 0)),
            pl.BlockSpec((BN, 1), lambda i: (i, 0)),
        ],
        out_specs=[
            pl.BlockSpec((BN, 16), lambda i: (i, 0)),
            pl.BlockSpec((BN, 16), lambda i: (i, 0)),
        ],
        out_shape=[
            jax.ShapeDtypeStruct((n, 16), jnp.float32),
            jax.ShapeDtypeStruct((n, 16), jnp.float32),
        ],
    )(acc2, s2, dinv)


def kernel(x, edge_index, edge_attr, W_dense, b_dense, W_mu, b_mu,
           W_logstd, b_logstd):
    n = x.shape[0]
    e = edge_attr.shape[0]
    e_pad = ((e + 32 * CHUNK - 1) // (32 * CHUNK)) * (32 * CHUNK)
    pad = e_pad - e

    src = jnp.concatenate([edge_index[0], jnp.zeros((pad,), jnp.int32)])
    dst = jnp.concatenate([edge_index[1], jnp.zeros((pad,), jnp.int32)])
    ewp = jnp.concatenate([edge_attr, jnp.zeros((pad,), jnp.float32)])
    src2d = src.reshape(-1, SUB)
    dst2d = dst.reshape(-1, SUB)

    zn = jnp.zeros((n,), jnp.float32)
    zn16 = jnp.zeros((n, 16), jnp.float32)

    dp = _deg_kernel(n, e_pad, dst2d, ewp, zn)
    dpt = dp.T

    wdt = W_dense.T
    wct = jnp.concatenate([W_mu, W_logstd], axis=0).T
    bd = b_dense.reshape(1, -1)
    bcat = jnp.stack([b_mu, b_logstd])

    g2, s2, dinv = _dense_kernel(x, wdt, bd, wct, bcat, dpt)
    g2r = g2.reshape(2 * n, 16)

    acc2 = _agg_kernel(n, e_pad, g2r, src2d, dst2d, ewp, zn16)

    mu, logstd = _final_kernel(acc2, s2, dinv)
    return (mu, logstd)


# speedup vs baseline: 30.4651x; 1.3035x over previous
import functools
import jax
import jax.numpy as jnp
from jax import lax
from jax.experimental import pallas as pl
from jax.experimental.pallas import tpu as pltpu
from jax.experimental.pallas import tpu_sc as plsc

CHUNK = 2048
SUB = 128
NSUB = CHUNK // SUB
CHUNK_B = 256
NSUB_B = CHUNK_B // SUB
NBUF = 4
BN = 2000


def _deg_kernel(n, e_pad, dst2d, ewp, zn):
    mesh = plsc.VectorSubcoreMesh(core_axis_name="c", subcore_axis_name="s")
    per_w = e_pad // 32
    n_chunks = per_w // CHUNK

    @functools.partial(
        pl.kernel,
        out_type=jax.ShapeDtypeStruct((2, n), jnp.float32),
        mesh=mesh,
        scratch_types=[
            pltpu.VMEM((NSUB, SUB), jnp.int32),
            pltpu.VMEM((CHUNK,), jnp.float32),
            pltpu.VMEM_SHARED((n,), jnp.float32),
        ],
    )
    def body(dst_hbm, ew_hbm, z_hbm, out_hbm, dv, wv, deg_sh):
        c = lax.axis_index("c")
        s = lax.axis_index("s")
        wid = c * 16 + s

        @pl.when(s == 0)
        def _():
            pltpu.sync_copy(z_hbm, deg_sh)

        plsc.subcore_barrier()

        def chunk(t, carry):
            base = pl.multiple_of(wid * per_w + t * CHUNK, CHUNK)
            rbase = pl.multiple_of(base // SUB, NSUB)
            pltpu.sync_copy(dst_hbm.at[pl.ds(rbase, NSUB)], dv)
            pltpu.sync_copy(ew_hbm.at[pl.ds(base, CHUNK)], wv)
            for j in range(NSUB):
                pltpu.sync_copy(wv.at[pl.ds(j * SUB, SUB)],
                                deg_sh.at[dv.at[j]], add=True)
            return carry

        lax.fori_loop(0, n_chunks, chunk, 0)
        plsc.subcore_barrier()

        @pl.when(s == 0)
        def _():
            pltpu.sync_copy(deg_sh, out_hbm.at[c])

    return body(dst2d, ewp, zn)


def _agg_kernel(n, e_pad, g2r, src2d, dst2d, ewp):
    mesh = plsc.VectorSubcoreMesh(core_axis_name="c", subcore_axis_name="s")
    per_t = e_pad // 16
    n_chunks = per_t // CHUNK_B
    n_quads = n_chunks // NBUF
    rows_a = ((n // 16 + 7) // 8) * 8
    nz_full = rows_a // CHUNK_B
    rem_lo = rows_a - nz_full * CHUNK_B
    rows_last = n - 15 * rows_a
    rem_hi = rows_last - nz_full * CHUNK_B

    scratch = []
    for _ in range(NBUF):
        scratch += [
            pltpu.VMEM((NSUB_B, SUB), jnp.int32),
            pltpu.VMEM((NSUB_B, SUB), jnp.int32),
            pltpu.VMEM((NSUB_B, SUB), jnp.int32),
            pltpu.VMEM((CHUNK_B,), jnp.float32),
            pltpu.VMEM((CHUNK_B, 16), jnp.float32),
            pltpu.SemaphoreType.DMA,
            pltpu.SemaphoreType.DMA,
        ]
    scratch.append(pltpu.VMEM_SHARED((n, 16), jnp.float32))

    @functools.partial(
        pl.kernel,
        out_type=jax.ShapeDtypeStruct((2, n, 16), jnp.float32),
        mesh=mesh,
        scratch_types=scratch,
        compiler_params=pltpu.CompilerParams(use_tc_tiling_on_sc=False),
    )
    def body(g_hbm, src_hbm, dst_hbm, ew_hbm, out_hbm, *refs):
        bufs = [refs[7 * b:7 * b + 7] for b in range(NBUF)]
        acc_sh = refs[7 * NBUF]
        c = lax.axis_index("c")
        s = lax.axis_index("s")
        coff = c * n

        z0 = bufs[0][4]

        def zr(r, cc):
            z0[r, :] = jnp.zeros((16,), jnp.float32)
            return cc
        lax.fori_loop(0, CHUNK_B, zr, 0)
        r0 = pl.multiple_of(s * rows_a, 8)
        for k in range(nz_full):
            pltpu.sync_copy(z0, acc_sh.at[pl.ds(r0 + k * CHUNK_B, CHUNK_B)])

        @pl.when(s < 15)
        def _():
            pltpu.sync_copy(z0.at[pl.ds(0, rem_lo)],
                            acc_sh.at[pl.ds(r0 + nz_full * CHUNK_B, rem_lo)])

        @pl.when(s == 15)
        def _():
            pltpu.sync_copy(z0.at[pl.ds(0, rem_hi)],
                            acc_sh.at[pl.ds(r0 + nz_full * CHUNK_B, rem_hi)])

        plsc.subcore_barrier()

        def load_idx(b, t):
            sv, iv, dv, wv = bufs[b][0], bufs[b][1], bufs[b][2], bufs[b][3]
            base = pl.multiple_of(s * per_t + t * CHUNK_B, CHUNK_B)
            rbase = pl.multiple_of(base // SUB, NSUB_B)
            pltpu.sync_copy(src_hbm.at[pl.ds(rbase, NSUB_B)], sv)
            pltpu.sync_copy(dst_hbm.at[pl.ds(rbase, NSUB_B)], dv)
            pltpu.sync_copy(ew_hbm.at[pl.ds(base, CHUNK_B)], wv)
            for j in range(NSUB_B):
                def addoff(k, cc):
                    iv[j, pl.ds(k * 16, 16)] = sv[j, pl.ds(k * 16, 16)] + coff
                    return cc
                lax.fori_loop(0, SUB // 16, addoff, 0)

        def fire_gathers(b):
            iv, rows, gsem = bufs[b][1], bufs[b][4], bufs[b][5]
            for j in range(NSUB_B):
                pltpu.async_copy(g_hbm.at[iv.at[j]],
                                 rows.at[pl.ds(j * SUB, SUB)], gsem)

        def wait_gathers(b):
            rows, gsem = bufs[b][4], bufs[b][5]
            pltpu.make_async_copy(g_hbm.at[pl.ds(0, CHUNK_B)], rows,
                                  gsem).wait()

        def scale(b):
            wv, rows = bufs[b][3], bufs[b][4]

            def grp(g, cc):
                ewv = wv[pl.ds(g * 16, 16)]
                for e in range(16):
                    be = ewv.at[jnp.full((16,), e, jnp.int32)].get(
                        mode="promise_in_bounds")
                    r = g * 16 + e
                    rows[r, :] = rows[r, :] * be
                return cc
            lax.fori_loop(0, CHUNK_B // 16, grp, 0)

        def fire_scatter(b):
            dv, rows, ssem = bufs[b][2], bufs[b][4], bufs[b][6]
            for j in range(NSUB_B):
                pltpu.async_copy(rows.at[pl.ds(j * SUB, SUB)],
                                 acc_sh.at[dv.at[j]], ssem, add=True)

        def wait_scatter(b):
            rows, ssem = bufs[b][4], bufs[b][6]
            pltpu.make_async_copy(g_hbm.at[pl.ds(0, CHUNK_B)], rows,
                                  ssem).wait()

        for b in range(NBUF - 1):
            load_idx(b, b)
            fire_gathers(b)

        def quad(i, cc):
            t0 = i * NBUF
            for b in range(NBUF):
                wait_gathers(b)
                scale(b)
                fire_scatter(b)
                bb = (b + NBUF - 1) % NBUF
                tref = t0 + b + NBUF - 1
                if b == 0:
                    @pl.when(i > 0)
                    def _():
                        wait_scatter(bb)
                    load_idx(bb, tref)
                    fire_gathers(bb)
                else:
                    @pl.when(i < n_quads - 1)
                    def _():
                        wait_scatter(bb)
                        load_idx(bb, tref)
                        fire_gathers(bb)
            return cc

        lax.fori_loop(0, n_quads, quad, 0)
        for b in range(NBUF):
            wait_scatter(b)

        plsc.subcore_barrier()

        @pl.when(s < 15)
        def _():
            pltpu.sync_copy(acc_sh.at[pl.ds(r0, rows_a)],
                            out_hbm.at[c, pl.ds(r0, rows_a)])

        @pl.when(s == 15)
        def _():
            pltpu.sync_copy(acc_sh.at[pl.ds(r0, rows_last)],
                            out_hbm.at[c, pl.ds(r0, rows_last)])

    return body(g2r, src2d, dst2d, ewp)


def _dense_kernel(x, wdt, bd, wct, bcat, dpt):
    n = x.shape[0]
    grid = n // BN

    def body(x_ref, wdt_ref, bd_ref, wct_ref, bcat_ref, dpt_ref,
             g2_ref, s2_ref, dinv_ref):
        xb = x_ref[...]
        h = jnp.maximum(
            lax.dot_general(xb, wdt_ref[...], (((1,), (0,)), ((), ())),
                            precision=lax.Precision.HIGHEST,
                            preferred_element_type=jnp.float32)
            + bd_ref[...], 0.0)
        h2 = lax.dot_general(h, wct_ref[...], (((1,), (0,)), ((), ())),
                             precision=lax.Precision.HIGHEST,
                             preferred_element_type=jnp.float32)
        dp = dpt_ref[...]
        deg = dp[:, 0] + dp[:, 1] + 1.0
        dinv = lax.rsqrt(deg)
        dinv_ref[...] = dinv[:, None]
        g = h2 * dinv[:, None]
        sself = h2 * (dinv * dinv)[:, None]
        g2_ref[0] = g[:, :16]
        g2_ref[1] = g[:, 16:]
        s2_ref[0] = sself[:, :16] + bcat_ref[0][None, :]
        s2_ref[1] = sself[:, 16:] + bcat_ref[1][None, :]

    return pl.pallas_call(
        body,
        grid=(grid,),
        in_specs=[
            pl.BlockSpec((BN, 128), lambda i: (i, 0)),
            pl.BlockSpec((128, 32), lambda i: (0, 0)),
            pl.BlockSpec((1, 32), lambda i: (0, 0)),
            pl.BlockSpec((32, 32), lambda i: (0, 0)),
            pl.BlockSpec((2, 16), lambda i: (0, 0)),
            pl.BlockSpec((BN, 2), lambda i: (i, 0)),
        ],
        out_specs=[
            pl.BlockSpec((2, BN, 16), lambda i: (0, i, 0)),
            pl.BlockSpec((2, BN, 16), lambda i: (0, i, 0)),
            pl.BlockSpec((BN, 1), lambda i: (i, 0)),
        ],
        out_shape=[
            jax.ShapeDtypeStruct((2, n, 16), jnp.float32),
            jax.ShapeDtypeStruct((2, n, 16), jnp.float32),
            jax.ShapeDtypeStruct((n, 1), jnp.float32),
        ],
    )(x, wdt, bd, wct, bcat, dpt)


def _final_kernel(acc2, s2, dinv):
    n = acc2.shape[1]
    grid = n // BN

    def body(a_ref, s_ref, d_ref, mu_ref, ls_ref):
        a = a_ref[...]
        sv = s_ref[...]
        d = d_ref[...]
        mu_ref[...] = a[0] * d + sv[0]
        ls_ref[...] = a[1] * d + sv[1]

    return pl.pallas_call(
        body,
        grid=(grid,),
        in_specs=[
            pl.BlockSpec((2, BN, 16), lambda i: (0, i, 0)),
            pl.BlockSpec((2, BN, 16), lambda i: (0, i, 0)),
            pl.BlockSpec((BN, 1), lambda i: (i, 0)),
        ],
        out_specs=[
            pl.BlockSpec((BN, 16), lambda i: (i, 0)),
            pl.BlockSpec((BN, 16), lambda i: (i, 0)),
        ],
        out_shape=[
            jax.ShapeDtypeStruct((n, 16), jnp.float32),
            jax.ShapeDtypeStruct((n, 16), jnp.float32),
        ],
    )(acc2, s2, dinv)


def kernel(x, edge_index, edge_attr, W_dense, b_dense, W_mu, b_mu,
           W_logstd, b_logstd):
    n = x.shape[0]
    e = edge_attr.shape[0]
    e_pad = ((e + 32 * CHUNK - 1) // (32 * CHUNK)) * (32 * CHUNK)
    pad = e_pad - e

    src = jnp.concatenate([edge_index[0], jnp.zeros((pad,), jnp.int32)])
    dst = jnp.concatenate([edge_index[1], jnp.zeros((pad,), jnp.int32)])
    ewp = jnp.concatenate([edge_attr, jnp.zeros((pad,), jnp.float32)])
    src2d = src.reshape(-1, SUB)
    dst2d = dst.reshape(-1, SUB)

    zn = jnp.zeros((n,), jnp.float32)

    dp = _deg_kernel(n, e_pad, dst2d, ewp, zn)
    dpt = dp.T

    wdt = W_dense.T
    wct = jnp.concatenate([W_mu, W_logstd], axis=0).T
    bd = b_dense.reshape(1, -1)
    bcat = jnp.stack([b_mu, b_logstd])

    g2, s2, dinv = _dense_kernel(x, wdt, bd, wct, bcat, dpt)
    g2r = g2.reshape(2 * n, 16)

    acc2 = _agg_kernel(n, e_pad, g2r, src2d, dst2d, ewp)

    mu, logstd = _final_kernel(acc2, s2, dinv)
    return (mu, logstd)
